# baseline (speedup 1.0000x reference)
import functools

import numpy as np
import jax
import jax.numpy as jnp
from jax import lax
from jax.experimental import pallas as pl
from jax.experimental.pallas import tpu as pltpu
from jax.experimental.pallas import tpu_sc as plsc

_D = 256
_M = 8
_DV = 32
_L = 4
_K = 4
_LK = _L * _K
_HW = [(64, 64), (32, 32), (16, 16), (8, 8)]
_N = 4
_Q = 5440
_S = 5440
_QB = 320
_NQB = _Q // _QB
_RB = 256
_QC = 2
_NCH = (_Q // 8) // _QC

def _level_consts():
    lane = lax.broadcasted_iota(jnp.int32, (1, _M * _LK), 1)
    lk = lane % _LK
    lev = lk // _K
    wi = jnp.int32(64) >> lev
    basev = jnp.where(lev == 0, 0,
                      jnp.where(lev == 1, 4096,
                                jnp.where(lev == 2, 5120, 5376)))
    m_idx = lane // _LK
    return wi.astype(jnp.float32), wi, basev, m_idx


def _matmul_bias_body(x_ref, w_ref, b_ref, o_ref):
    o_ref[...] = jnp.dot(
        x_ref[...], w_ref[...], preferred_element_type=jnp.float32,
        precision=lax.Precision.HIGHEST) + b_ref[...]


def _matmul_bias(x, w, b):
    rows, kdim = x.shape
    cols = w.shape[1]
    return pl.pallas_call(
        _matmul_bias_body,
        grid=(rows // _RB,),
        in_specs=[
            pl.BlockSpec((_RB, kdim), lambda i: (i, 0)),
            pl.BlockSpec((kdim, cols), lambda i: (0, 0)),
            pl.BlockSpec((1, cols), lambda i: (0, 0)),
        ],
        out_specs=pl.BlockSpec((_RB, cols), lambda i: (i, 0)),
        out_shape=jax.ShapeDtypeStruct((rows, cols), jnp.float32),
    )(x, w, b.reshape(1, cols))


def _params_body(q_ref, g_ref, wa_ref, ba_ref, wox_ref, box_ref, woy_ref,
                 boy_ref, gmat_ref, idx_ref, w_ref):
    n = pl.program_id(0)
    qs = q_ref[0]
    geo = g_ref[0]
    logits = jnp.dot(qs, wa_ref[...], preferred_element_type=jnp.float32,
                     precision=lax.Precision.HIGHEST) + ba_ref[...]
    e = jnp.exp(logits - jnp.max(logits, axis=-1, keepdims=True))
    s = jnp.dot(e, gmat_ref[...], preferred_element_type=jnp.float32,
                precision=lax.Precision.HIGHEST)
    attn = e / s
    pxo = jnp.dot(qs, wox_ref[...], preferred_element_type=jnp.float32,
                  precision=lax.Precision.HIGHEST) + box_ref[...]
    pyo = jnp.dot(qs, woy_ref[...], preferred_element_type=jnp.float32,
                  precision=lax.Precision.HIGHEST) + boy_ref[...]
    px = geo[:, 0:1] + pxo * (geo[:, 2:3] * (0.5 / _K))
    py = geo[:, 1:2] + pyo * (geo[:, 3:4] * (0.5 / _K))
    wf, wi, basev, m_idx = _level_consts()
    x = px * wf - 0.5
    y = py * wf - 0.5
    x0 = jnp.floor(x)
    y0 = jnp.floor(y)
    fx = x - x0
    fy = y - y0

    def corner(xi, yi, wgt):
        vx = (xi >= 0.0) & (xi <= wf - 1.0)
        vy = (yi >= 0.0) & (yi <= wf - 1.0)
        xc = jnp.clip(xi, 0.0, wf - 1.0).astype(jnp.int32)
        yc = jnp.clip(yi, 0.0, wf - 1.0).astype(jnp.int32)
        s_flat = basev + yc * wi + xc
        row = (n * _S + s_flat) * _M + m_idx
        wcomb = wgt * attn * vx.astype(jnp.float32) * vy.astype(jnp.float32)
        return row, wcomb

    r00, w00 = corner(x0, y0, (1.0 - fx) * (1.0 - fy))
    r10, w10 = corner(x0 + 1.0, y0, fx * (1.0 - fy))
    r01, w01 = corner(x0, y0 + 1.0, (1.0 - fx) * fy)
    r11, w11 = corner(x0 + 1.0, y0 + 1.0, fx * fy)
    idx_ref[0, 0] = jnp.concatenate([r00, r10, r01, r11], axis=-1)
    w_ref[0, 0] = jnp.concatenate([w00, w10, w01, w11], axis=-1)


def _sampling_params(queries, query_geometries, w_attn, b_attn, w_off, b_off):
    mlk = _M * _LK
    gmat = (lax.broadcasted_iota(jnp.int32, (mlk, mlk), 0) // _LK
            == lax.broadcasted_iota(jnp.int32, (mlk, mlk), 1) // _LK
            ).astype(jnp.float32)
    idx, wgt = pl.pallas_call(
        _params_body,
        grid=(_N, _NQB),
        in_specs=[
            pl.BlockSpec((1, _QB, _D), lambda n, qb: (n, qb, 0)),
            pl.BlockSpec((1, _QB, 4), lambda n, qb: (n, qb, 0)),
            pl.BlockSpec((_D, mlk), lambda n, qb: (0, 0)),
            pl.BlockSpec((1, mlk), lambda n, qb: (0, 0)),
            pl.BlockSpec((_D, mlk), lambda n, qb: (0, 0)),
            pl.BlockSpec((1, mlk), lambda n, qb: (0, 0)),
            pl.BlockSpec((_D, mlk), lambda n, qb: (0, 0)),
            pl.BlockSpec((1, mlk), lambda n, qb: (0, 0)),
            pl.BlockSpec((mlk, mlk), lambda n, qb: (0, 0)),
        ],
        out_specs=[
            pl.BlockSpec((1, 1, _QB, 4 * mlk), lambda n, qb: (n, qb, 0, 0)),
            pl.BlockSpec((1, 1, _QB, 4 * mlk), lambda n, qb: (n, qb, 0, 0)),
        ],
        out_shape=[
            jax.ShapeDtypeStruct((_N, _NQB, _QB, 4 * mlk), jnp.int32),
            jax.ShapeDtypeStruct((_N, _NQB, _QB, 4 * mlk), jnp.float32),
        ],
    )(queries, query_geometries, w_attn, b_attn.reshape(1, -1),
      w_off[:, 0::2], b_off[0::2].reshape(1, -1),
      w_off[:, 1::2], b_off[1::2].reshape(1, -1), gmat)

    return (idx.reshape(_N, _Q * 4, _M * _LK),
            wgt.reshape(_N, _Q * 4, _M * _LK))


def _sc_gather(table, cidx, cw):
    mesh = plsc.VectorSubcoreMesh(core_axis_name="c", subcore_axis_name="s")
    rpc = _QC * 4
    qpw = _Q // 8

    @functools.partial(
        pl.kernel, mesh=mesh,
        compiler_params=pltpu.CompilerParams(use_tc_tiling_on_sc=False),
        out_type=jax.ShapeDtypeStruct((_N, _Q, _M, _DV), jnp.float32),
        scratch_types=[
            pltpu.VMEM((rpc, 128), jnp.int32),
            pltpu.VMEM((rpc, 128), jnp.int32),
            pltpu.VMEM((rpc, 128), jnp.float32),
            pltpu.VMEM((rpc, 128), jnp.float32),
            pltpu.VMEM((rpc, 128, _DV), jnp.float32),
            pltpu.VMEM((rpc, 128, _DV), jnp.float32),
            pltpu.VMEM((_QC, _M, _DV), jnp.float32),
            pltpu.SemaphoreType.DMA,
            pltpu.SemaphoreType.DMA,
        ])
    def body(table_h, cidx_h, cw_h, agg_h, idx_a, idx_b, w_a, w_b, rows_a,
             rows_b, out_v, sem_a, sem_b):
        wid = lax.axis_index("s") * 2 + lax.axis_index("c")
        n = wid // 8
        q0 = lax.rem(wid, 8) * qpw

        def fire(jb, idx_v, w_v, rows_v, sem):
            r0 = (q0 + jb * _QC) * 4
            pltpu.sync_copy(cidx_h.at[n, pl.ds(r0, rpc)], idx_v)
            pltpu.sync_copy(cw_h.at[n, pl.ds(r0, rpc)], w_v)
            for j in range(rpc):
                pltpu.async_copy(table_h.at[idx_v.at[j]], rows_v.at[j], sem)

        def drain(rows_v, sem):
            for j in range(rpc):
                pltpu.make_async_copy(
                    table_h.at[pl.ds(0, 128)], rows_v.at[j], sem).wait()

        def compute(jb, w_v, rows_v):
            for q in range(_QC):
                def mloop(mi, c2, q=q):
                    cb = mi * 16
                    p0 = [jnp.zeros((16,), jnp.float32) for _ in range(4)]
                    p1 = [jnp.zeros((16,), jnp.float32) for _ in range(4)]
                    for c4 in range(4):
                        r = q * 4 + c4
                        wvec = w_v[r, pl.ds(cb, 16)]
                        for c16 in range(16):
                            wsc = wvec[c16]
                            p0[c4] = p0[c4] + wsc * rows_v[r, cb + c16, pl.ds(0, 16)]
                            p1[c4] = p1[c4] + wsc * rows_v[r, cb + c16, pl.ds(16, 16)]
                    out_v[q, mi, pl.ds(0, 16)] = (p0[0] + p0[1]) + (p0[2] + p0[3])
                    out_v[q, mi, pl.ds(16, 16)] = (p1[0] + p1[1]) + (p1[2] + p1[3])
                    return c2

                lax.fori_loop(0, _M, mloop, 0, unroll=4)
            pltpu.sync_copy(out_v, agg_h.at[n, pl.ds(q0 + jb * _QC, _QC)])

        fire(0, idx_a, w_a, rows_a, sem_a)

        def loop2(i, carry):
            jb0 = i * 2
            jb1 = jb0 + 1
            fire(jb1, idx_b, w_b, rows_b, sem_b)
            drain(rows_a, sem_a)
            compute(jb0, w_a, rows_a)

            @pl.when(jb1 + 1 < _NCH)
            def _():
                fire(jb1 + 1, idx_a, w_a, rows_a, sem_a)

            drain(rows_b, sem_b)
            compute(jb1, w_b, rows_b)
            return carry

        lax.fori_loop(0, _NCH // 2, loop2, 0)

    return body(table, cidx, cw)


def kernel(queries, query_geometries, value_inputs, value_pyramid_hw_sizes,
           W_off, b_off, W_attn, b_attn, W_val, b_val, W_out, b_out):
    del value_pyramid_hw_sizes
    proj = _matmul_bias(value_inputs.reshape(_N * _S, _D), W_val, b_val)
    table = proj.reshape(_N * _S * _M, _DV)
    cidx, cw = _sampling_params(queries, query_geometries, W_attn, b_attn,
                                W_off, b_off)
    agg = _sc_gather(table, cidx, cw)
    out = _matmul_bias(agg.reshape(_N * _Q, _M * _DV), W_out, b_out)
    return out.reshape(_N, _Q, _D)


# speedup vs baseline: 1.1907x; 1.1907x over previous
import functools

import numpy as np
import jax
import jax.numpy as jnp
from jax import lax
from jax.experimental import pallas as pl
from jax.experimental.pallas import tpu as pltpu
from jax.experimental.pallas import tpu_sc as plsc

_D = 256
_M = 8
_DV = 32
_L = 4
_K = 4
_LK = _L * _K
_HW = [(64, 64), (32, 32), (16, 16), (8, 8)]
_N = 4
_Q = 5440
_S = 5440
_QB = 320
_NQB = _Q // _QB
_RB = 256
_QC = 2
_NCH = (_Q // 8) // _QC

def _level_consts():
    lane = lax.broadcasted_iota(jnp.int32, (1, _M * _LK), 1)
    lk = lane % _LK
    lev = lk // _K
    wi = jnp.int32(64) >> lev
    basev = jnp.where(lev == 0, 0,
                      jnp.where(lev == 1, 4096,
                                jnp.where(lev == 2, 5120, 5376)))
    m_idx = lane // _LK
    return wi.astype(jnp.float32), wi, basev, m_idx


def _matmul_bias_body(x_ref, w_ref, b_ref, o_ref):
    o_ref[...] = jnp.dot(
        x_ref[...], w_ref[...], preferred_element_type=jnp.float32,
        precision=lax.Precision.HIGHEST) + b_ref[...]


def _matmul_bias(x, w, b):
    rows, kdim = x.shape
    cols = w.shape[1]
    return pl.pallas_call(
        _matmul_bias_body,
        grid=(rows // _RB,),
        in_specs=[
            pl.BlockSpec((_RB, kdim), lambda i: (i, 0)),
            pl.BlockSpec((kdim, cols), lambda i: (0, 0)),
            pl.BlockSpec((1, cols), lambda i: (0, 0)),
        ],
        out_specs=pl.BlockSpec((_RB, cols), lambda i: (i, 0)),
        out_shape=jax.ShapeDtypeStruct((rows, cols), jnp.float32),
    )(x, w, b.reshape(1, cols))


def _params_body(q_ref, g_ref, wa_ref, ba_ref, wox_ref, box_ref, woy_ref,
                 boy_ref, gmat_ref, idx_ref, w_ref):
    n = pl.program_id(0)
    qs = q_ref[0]
    geo = g_ref[0]
    logits = jnp.dot(qs, wa_ref[...], preferred_element_type=jnp.float32,
                     precision=lax.Precision.HIGHEST) + ba_ref[...]
    e = jnp.exp(logits - jnp.max(logits, axis=-1, keepdims=True))
    s = jnp.dot(e, gmat_ref[...], preferred_element_type=jnp.float32,
                precision=lax.Precision.HIGHEST)
    attn = e / s
    pxo = jnp.dot(qs, wox_ref[...], preferred_element_type=jnp.float32,
                  precision=lax.Precision.HIGHEST) + box_ref[...]
    pyo = jnp.dot(qs, woy_ref[...], preferred_element_type=jnp.float32,
                  precision=lax.Precision.HIGHEST) + boy_ref[...]
    px = geo[:, 0:1] + pxo * (geo[:, 2:3] * (0.5 / _K))
    py = geo[:, 1:2] + pyo * (geo[:, 3:4] * (0.5 / _K))
    wf, wi, basev, m_idx = _level_consts()
    x = px * wf - 0.5
    y = py * wf - 0.5
    x0 = jnp.floor(x)
    y0 = jnp.floor(y)
    fx = x - x0
    fy = y - y0

    def corner(xi, yi, wgt):
        vx = (xi >= 0.0) & (xi <= wf - 1.0)
        vy = (yi >= 0.0) & (yi <= wf - 1.0)
        xc = jnp.clip(xi, 0.0, wf - 1.0).astype(jnp.int32)
        yc = jnp.clip(yi, 0.0, wf - 1.0).astype(jnp.int32)
        s_flat = basev + yc * wi + xc
        row = (n * _S + s_flat) * _M + m_idx
        wcomb = wgt * attn * vx.astype(jnp.float32) * vy.astype(jnp.float32)
        return row, wcomb

    r00, w00 = corner(x0, y0, (1.0 - fx) * (1.0 - fy))
    r10, w10 = corner(x0 + 1.0, y0, fx * (1.0 - fy))
    r01, w01 = corner(x0, y0 + 1.0, (1.0 - fx) * fy)
    r11, w11 = corner(x0 + 1.0, y0 + 1.0, fx * fy)
    idx_ref[0, 0] = jnp.concatenate([r00, r10, r01, r11], axis=-1)
    w_ref[0, 0] = jnp.concatenate([w00, w10, w01, w11], axis=-1)


def _sampling_params(queries, query_geometries, w_attn, b_attn, w_off, b_off):
    mlk = _M * _LK
    gmat = (lax.broadcasted_iota(jnp.int32, (mlk, mlk), 0) // _LK
            == lax.broadcasted_iota(jnp.int32, (mlk, mlk), 1) // _LK
            ).astype(jnp.float32)
    idx, wgt = pl.pallas_call(
        _params_body,
        grid=(_N, _NQB),
        in_specs=[
            pl.BlockSpec((1, _QB, _D), lambda n, qb: (n, qb, 0)),
            pl.BlockSpec((1, _QB, 4), lambda n, qb: (n, qb, 0)),
            pl.BlockSpec((_D, mlk), lambda n, qb: (0, 0)),
            pl.BlockSpec((1, mlk), lambda n, qb: (0, 0)),
            pl.BlockSpec((_D, mlk), lambda n, qb: (0, 0)),
            pl.BlockSpec((1, mlk), lambda n, qb: (0, 0)),
            pl.BlockSpec((_D, mlk), lambda n, qb: (0, 0)),
            pl.BlockSpec((1, mlk), lambda n, qb: (0, 0)),
            pl.BlockSpec((mlk, mlk), lambda n, qb: (0, 0)),
        ],
        out_specs=[
            pl.BlockSpec((1, 1, _QB, 4 * mlk), lambda n, qb: (n, qb, 0, 0)),
            pl.BlockSpec((1, 1, _QB, 4 * mlk), lambda n, qb: (n, qb, 0, 0)),
        ],
        out_shape=[
            jax.ShapeDtypeStruct((_N, _NQB, _QB, 4 * mlk), jnp.int32),
            jax.ShapeDtypeStruct((_N, _NQB, _QB, 4 * mlk), jnp.float32),
        ],
    )(queries, query_geometries, w_attn, b_attn.reshape(1, -1),
      w_off[:, 0::2], b_off[0::2].reshape(1, -1),
      w_off[:, 1::2], b_off[1::2].reshape(1, -1), gmat)

    return (idx.reshape(_N, _Q * 4, _M * _LK),
            wgt.reshape(_N, _Q * 4, _M * _LK))


def _sc_gather(table, cidx, cw):
    mesh = plsc.VectorSubcoreMesh(core_axis_name="c", subcore_axis_name="s")
    rpc = _QC * 4
    qpw = _Q // 8

    @functools.partial(
        pl.kernel, mesh=mesh,
        compiler_params=pltpu.CompilerParams(use_tc_tiling_on_sc=False),
        out_type=jax.ShapeDtypeStruct((_N, _Q, _M, _DV), jnp.float32),
        scratch_types=[
            pltpu.VMEM((rpc, 128), jnp.int32),
            pltpu.VMEM((rpc, 128), jnp.int32),
            pltpu.VMEM((rpc, 128), jnp.float32),
            pltpu.VMEM((rpc, 128), jnp.float32),
            pltpu.VMEM((rpc, 128, _DV), jnp.float32),
            pltpu.VMEM((rpc, 128, _DV), jnp.float32),
            pltpu.VMEM((_QC, _M, _DV), jnp.float32),
            pltpu.SemaphoreType.DMA,
            pltpu.SemaphoreType.DMA,
        ])
    def body(table_h, cidx_h, cw_h, agg_h, idx_a, idx_b, w_a, w_b, rows_a,
             rows_b, out_v, sem_a, sem_b):
        wid = lax.axis_index("s") * 2 + lax.axis_index("c")
        n = wid // 8
        q0 = lax.rem(wid, 8) * qpw

        def fire(jb, idx_v, w_v, rows_v, sem):
            r0 = (q0 + jb * _QC) * 4
            pltpu.sync_copy(cidx_h.at[n, pl.ds(r0, rpc)], idx_v)
            pltpu.sync_copy(cw_h.at[n, pl.ds(r0, rpc)], w_v)
            for j in range(rpc):
                pltpu.async_copy(table_h.at[idx_v.at[j]], rows_v.at[j], sem)

        def drain(rows_v, sem):
            for j in range(rpc):
                pltpu.make_async_copy(
                    table_h.at[pl.ds(0, 128)], rows_v.at[j], sem).wait()

        def compute(jb, w_v, rows_v):
            for q in range(_QC):
                def mloop(mi, c2, q=q):
                    cb = mi * 16
                    p0 = [jnp.zeros((16,), jnp.float32) for _ in range(4)]
                    p1 = [jnp.zeros((16,), jnp.float32) for _ in range(4)]
                    for c4 in range(4):
                        r = q * 4 + c4
                        wvec = w_v[r, pl.ds(cb, 16)]
                        for c16 in range(16):
                            wsc = wvec[c16]
                            p0[c4] = p0[c4] + wsc * rows_v[r, cb + c16, pl.ds(0, 16)]
                            p1[c4] = p1[c4] + wsc * rows_v[r, cb + c16, pl.ds(16, 16)]
                    out_v[q, mi, pl.ds(0, 16)] = (p0[0] + p0[1]) + (p0[2] + p0[3])
                    out_v[q, mi, pl.ds(16, 16)] = (p1[0] + p1[1]) + (p1[2] + p1[3])
                    return c2

                lax.fori_loop(0, _M, mloop, 0, unroll=2)
            pltpu.sync_copy(out_v, agg_h.at[n, pl.ds(q0 + jb * _QC, _QC)])

        fire(0, idx_a, w_a, rows_a, sem_a)

        def loop2(i, carry):
            jb0 = i * 2
            jb1 = jb0 + 1
            fire(jb1, idx_b, w_b, rows_b, sem_b)
            drain(rows_a, sem_a)
            compute(jb0, w_a, rows_a)

            @pl.when(jb1 + 1 < _NCH)
            def _():
                fire(jb1 + 1, idx_a, w_a, rows_a, sem_a)

            drain(rows_b, sem_b)
            compute(jb1, w_b, rows_b)
            return carry

        lax.fori_loop(0, _NCH // 2, loop2, 0)

    return body(table, cidx, cw)


def kernel(queries, query_geometries, value_inputs, value_pyramid_hw_sizes,
           W_off, b_off, W_attn, b_attn, W_val, b_val, W_out, b_out):
    del value_pyramid_hw_sizes
    proj = _matmul_bias(value_inputs.reshape(_N * _S, _D), W_val, b_val)
    table = proj.reshape(_N * _S * _M, _DV)
    cidx, cw = _sampling_params(queries, query_geometries, W_attn, b_attn,
                                W_off, b_off)
    agg = _sc_gather(table, cidx, cw)
    out = _matmul_bias(agg.reshape(_N * _Q, _M * _DV), W_out, b_out)
    return out.reshape(_N, _Q, _D)


# speedup vs baseline: 1.1965x; 1.0048x over previous
import functools

import numpy as np
import jax
import jax.numpy as jnp
from jax import lax
from jax.experimental import pallas as pl
from jax.experimental.pallas import tpu as pltpu
from jax.experimental.pallas import tpu_sc as plsc

_D = 256
_M = 8
_DV = 32
_L = 4
_K = 4
_LK = _L * _K
_HW = [(64, 64), (32, 32), (16, 16), (8, 8)]
_N = 4
_Q = 5440
_S = 5440
_QB = 320
_NQB = _Q // _QB
_RB = 256
_QC = 2
_NCH = (_Q // 8) // _QC

def _level_consts():
    lane = lax.broadcasted_iota(jnp.int32, (1, _M * _LK), 1)
    lk = lane % _LK
    lev = lk // _K
    wi = jnp.int32(64) >> lev
    basev = jnp.where(lev == 0, 0,
                      jnp.where(lev == 1, 4096,
                                jnp.where(lev == 2, 5120, 5376)))
    m_idx = lane // _LK
    return wi.astype(jnp.float32), wi, basev, m_idx


def _matmul_bias_body(x_ref, w_ref, b_ref, o_ref):
    o_ref[...] = jnp.dot(
        x_ref[...], w_ref[...], preferred_element_type=jnp.float32,
        precision=lax.Precision.HIGHEST) + b_ref[...]


def _matmul_bias(x, w, b):
    rows, kdim = x.shape
    cols = w.shape[1]
    return pl.pallas_call(
        _matmul_bias_body,
        grid=(rows // _RB,),
        in_specs=[
            pl.BlockSpec((_RB, kdim), lambda i: (i, 0)),
            pl.BlockSpec((kdim, cols), lambda i: (0, 0)),
            pl.BlockSpec((1, cols), lambda i: (0, 0)),
        ],
        out_specs=pl.BlockSpec((_RB, cols), lambda i: (i, 0)),
        out_shape=jax.ShapeDtypeStruct((rows, cols), jnp.float32),
    )(x, w, b.reshape(1, cols))


def _params_body(q_ref, g_ref, wa_ref, ba_ref, wox_ref, box_ref, woy_ref,
                 boy_ref, gmat_ref, idx_ref, w_ref):
    n = pl.program_id(0)
    qs = q_ref[0]
    geo = g_ref[0]
    logits = jnp.dot(qs, wa_ref[...], preferred_element_type=jnp.float32,
                     precision=lax.Precision.HIGHEST) + ba_ref[...]
    e = jnp.exp(logits - jnp.max(logits, axis=-1, keepdims=True))
    s = jnp.dot(e, gmat_ref[...], preferred_element_type=jnp.float32,
                precision=lax.Precision.HIGHEST)
    attn = e / s
    pxo = jnp.dot(qs, wox_ref[...], preferred_element_type=jnp.float32,
                  precision=lax.Precision.HIGHEST) + box_ref[...]
    pyo = jnp.dot(qs, woy_ref[...], preferred_element_type=jnp.float32,
                  precision=lax.Precision.HIGHEST) + boy_ref[...]
    px = geo[:, 0:1] + pxo * (geo[:, 2:3] * (0.5 / _K))
    py = geo[:, 1:2] + pyo * (geo[:, 3:4] * (0.5 / _K))
    wf, wi, basev, m_idx = _level_consts()
    x = px * wf - 0.5
    y = py * wf - 0.5
    x0 = jnp.floor(x)
    y0 = jnp.floor(y)
    fx = x - x0
    fy = y - y0

    def corner(xi, yi, wgt):
        vx = (xi >= 0.0) & (xi <= wf - 1.0)
        vy = (yi >= 0.0) & (yi <= wf - 1.0)
        xc = jnp.clip(xi, 0.0, wf - 1.0).astype(jnp.int32)
        yc = jnp.clip(yi, 0.0, wf - 1.0).astype(jnp.int32)
        s_flat = basev + yc * wi + xc
        row = (n * _S + s_flat) * _M + m_idx
        wcomb = wgt * attn * vx.astype(jnp.float32) * vy.astype(jnp.float32)
        return row, wcomb

    r00, w00 = corner(x0, y0, (1.0 - fx) * (1.0 - fy))
    r10, w10 = corner(x0 + 1.0, y0, fx * (1.0 - fy))
    r01, w01 = corner(x0, y0 + 1.0, (1.0 - fx) * fy)
    r11, w11 = corner(x0 + 1.0, y0 + 1.0, fx * fy)
    idx_ref[0, 0] = jnp.concatenate([r00, r10, r01, r11], axis=-1)
    w_ref[0, 0] = jnp.concatenate([w00, w10, w01, w11], axis=-1)


def _sampling_params(queries, query_geometries, w_attn, b_attn, w_off, b_off):
    mlk = _M * _LK
    gmat = (lax.broadcasted_iota(jnp.int32, (mlk, mlk), 0) // _LK
            == lax.broadcasted_iota(jnp.int32, (mlk, mlk), 1) // _LK
            ).astype(jnp.float32)
    idx, wgt = pl.pallas_call(
        _params_body,
        grid=(_N, _NQB),
        in_specs=[
            pl.BlockSpec((1, _QB, _D), lambda n, qb: (n, qb, 0)),
            pl.BlockSpec((1, _QB, 4), lambda n, qb: (n, qb, 0)),
            pl.BlockSpec((_D, mlk), lambda n, qb: (0, 0)),
            pl.BlockSpec((1, mlk), lambda n, qb: (0, 0)),
            pl.BlockSpec((_D, mlk), lambda n, qb: (0, 0)),
            pl.BlockSpec((1, mlk), lambda n, qb: (0, 0)),
            pl.BlockSpec((_D, mlk), lambda n, qb: (0, 0)),
            pl.BlockSpec((1, mlk), lambda n, qb: (0, 0)),
            pl.BlockSpec((mlk, mlk), lambda n, qb: (0, 0)),
        ],
        out_specs=[
            pl.BlockSpec((1, 1, _QB, 4 * mlk), lambda n, qb: (n, qb, 0, 0)),
            pl.BlockSpec((1, 1, _QB, 4 * mlk), lambda n, qb: (n, qb, 0, 0)),
        ],
        out_shape=[
            jax.ShapeDtypeStruct((_N, _NQB, _QB, 4 * mlk), jnp.int32),
            jax.ShapeDtypeStruct((_N, _NQB, _QB, 4 * mlk), jnp.float32),
        ],
    )(queries, query_geometries, w_attn, b_attn.reshape(1, -1),
      w_off[:, 0::2], b_off[0::2].reshape(1, -1),
      w_off[:, 1::2], b_off[1::2].reshape(1, -1), gmat)

    return (idx.reshape(_N, _Q * 4, _M * _LK),
            wgt.reshape(_N, _Q * 4, _M * _LK))


def _sc_gather(table, cidx, cw):
    mesh = plsc.VectorSubcoreMesh(core_axis_name="c", subcore_axis_name="s")
    rpc = _QC * 4
    qpw = _Q // 8

    @functools.partial(
        pl.kernel, mesh=mesh,
        compiler_params=pltpu.CompilerParams(use_tc_tiling_on_sc=False),
        out_type=jax.ShapeDtypeStruct((_N, _Q, _M, _DV), jnp.float32),
        scratch_types=[
            pltpu.VMEM((rpc, 128), jnp.int32),
            pltpu.VMEM((rpc, 128), jnp.int32),
            pltpu.VMEM((rpc, 128), jnp.float32),
            pltpu.VMEM((rpc, 128), jnp.float32),
            pltpu.VMEM((rpc, 128, _DV), jnp.float32),
            pltpu.VMEM((rpc, 128, _DV), jnp.float32),
            pltpu.VMEM((_QC, _M, _DV), jnp.float32),
            pltpu.SemaphoreType.DMA,
            pltpu.SemaphoreType.DMA,
        ])
    def body(table_h, cidx_h, cw_h, agg_h, idx_a, idx_b, w_a, w_b, rows_a,
             rows_b, out_v, sem_a, sem_b):
        wid = lax.axis_index("s") * 2 + lax.axis_index("c")
        n = wid // 8
        q0 = lax.rem(wid, 8) * qpw

        def fire(jb, idx_v, w_v, rows_v, sem):
            r0 = (q0 + jb * _QC) * 4
            pltpu.sync_copy(cidx_h.at[n, pl.ds(r0, rpc)], idx_v)
            pltpu.sync_copy(cw_h.at[n, pl.ds(r0, rpc)], w_v)
            for j in range(rpc):
                pltpu.async_copy(table_h.at[idx_v.at[j]], rows_v.at[j], sem)

        def drain(rows_v, sem):
            for j in range(rpc):
                pltpu.make_async_copy(
                    table_h.at[pl.ds(0, 128)], rows_v.at[j], sem).wait()

        def compute(jb, w_v, rows_v):
            for q in range(_QC):
                def mloop(mi, c2, q=q):
                    cb = mi * 16
                    acc0 = jnp.zeros((16,), jnp.float32)
                    acc1 = jnp.zeros((16,), jnp.float32)
                    for c4 in range(4):
                        r = q * 4 + c4
                        wvec = w_v[r, pl.ds(cb, 16)]
                        for c16 in range(16):
                            wsc = wvec[c16]
                            acc0 = acc0 + wsc * rows_v[r, cb + c16, pl.ds(0, 16)]
                            acc1 = acc1 + wsc * rows_v[r, cb + c16, pl.ds(16, 16)]
                    out_v[q, mi, pl.ds(0, 16)] = acc0
                    out_v[q, mi, pl.ds(16, 16)] = acc1
                    return c2

                lax.fori_loop(0, _M, mloop, 0, unroll=2)
            pltpu.sync_copy(out_v, agg_h.at[n, pl.ds(q0 + jb * _QC, _QC)])

        fire(0, idx_a, w_a, rows_a, sem_a)

        def loop2(i, carry):
            jb0 = i * 2
            jb1 = jb0 + 1
            fire(jb1, idx_b, w_b, rows_b, sem_b)
            drain(rows_a, sem_a)
            compute(jb0, w_a, rows_a)

            @pl.when(jb1 + 1 < _NCH)
            def _():
                fire(jb1 + 1, idx_a, w_a, rows_a, sem_a)

            drain(rows_b, sem_b)
            compute(jb1, w_b, rows_b)
            return carry

        lax.fori_loop(0, _NCH // 2, loop2, 0)

    return body(table, cidx, cw)


def kernel(queries, query_geometries, value_inputs, value_pyramid_hw_sizes,
           W_off, b_off, W_attn, b_attn, W_val, b_val, W_out, b_out):
    del value_pyramid_hw_sizes
    proj = _matmul_bias(value_inputs.reshape(_N * _S, _D), W_val, b_val)
    table = proj.reshape(_N * _S * _M, _DV)
    cidx, cw = _sampling_params(queries, query_geometries, W_attn, b_attn,
                                W_off, b_off)
    agg = _sc_gather(table, cidx, cw)
    out = _matmul_bias(agg.reshape(_N * _Q, _M * _DV), W_out, b_out)
    return out.reshape(_N, _Q, _D)


# speedup vs baseline: 1.1966x; 1.0001x over previous
import functools

import jax
import jax.numpy as jnp
from jax import lax
from jax.experimental import pallas as pl
from jax.experimental.pallas import tpu as pltpu
from jax.experimental.pallas import tpu_sc as plsc

_D = 256
_M = 8
_DV = 32
_L = 4
_K = 4
_LK = _L * _K
_HW = [(64, 64), (32, 32), (16, 16), (8, 8)]
_N = 4
_Q = 5440
_S = 5440
_QB = 320
_NQB = _Q // _QB
_RB = 256
_QC = 2
_NCH = (_Q // 8) // _QC

def _level_consts():
    lane = lax.broadcasted_iota(jnp.int32, (1, _M * _LK), 1)
    lk = lane % _LK
    lev = lk // _K
    wi = jnp.int32(64) >> lev
    basev = jnp.where(lev == 0, 0,
                      jnp.where(lev == 1, 4096,
                                jnp.where(lev == 2, 5120, 5376)))
    m_idx = lane // _LK
    return wi.astype(jnp.float32), wi, basev, m_idx


def _matmul_bias_body(x_ref, w_ref, b_ref, o_ref):
    o_ref[...] = jnp.dot(
        x_ref[...], w_ref[...], preferred_element_type=jnp.float32,
        precision=lax.Precision.HIGHEST) + b_ref[...]


def _matmul_bias(x, w, b):
    rows, kdim = x.shape
    cols = w.shape[1]
    return pl.pallas_call(
        _matmul_bias_body,
        grid=(rows // _RB,),
        in_specs=[
            pl.BlockSpec((_RB, kdim), lambda i: (i, 0)),
            pl.BlockSpec((kdim, cols), lambda i: (0, 0)),
            pl.BlockSpec((1, cols), lambda i: (0, 0)),
        ],
        out_specs=pl.BlockSpec((_RB, cols), lambda i: (i, 0)),
        out_shape=jax.ShapeDtypeStruct((rows, cols), jnp.float32),
    )(x, w, b.reshape(1, cols))


def _params_body(q_ref, g_ref, wa_ref, ba_ref, wox_ref, box_ref, woy_ref,
                 boy_ref, gmat_ref, idx_ref, w_ref):
    n = pl.program_id(0)
    qs = q_ref[0]
    geo = g_ref[0]
    logits = jnp.dot(qs, wa_ref[...], preferred_element_type=jnp.float32,
                     precision=lax.Precision.HIGHEST) + ba_ref[...]
    e = jnp.exp(logits - jnp.max(logits, axis=-1, keepdims=True))
    s = jnp.dot(e, gmat_ref[...], preferred_element_type=jnp.float32,
                precision=lax.Precision.HIGHEST)
    attn = e / s
    pxo = jnp.dot(qs, wox_ref[...], preferred_element_type=jnp.float32,
                  precision=lax.Precision.HIGHEST) + box_ref[...]
    pyo = jnp.dot(qs, woy_ref[...], preferred_element_type=jnp.float32,
                  precision=lax.Precision.HIGHEST) + boy_ref[...]
    px = geo[:, 0:1] + pxo * (geo[:, 2:3] * (0.5 / _K))
    py = geo[:, 1:2] + pyo * (geo[:, 3:4] * (0.5 / _K))
    wf, wi, basev, m_idx = _level_consts()
    x = px * wf - 0.5
    y = py * wf - 0.5
    x0 = jnp.floor(x)
    y0 = jnp.floor(y)
    fx = x - x0
    fy = y - y0

    def corner(xi, yi, wgt):
        vx = (xi >= 0.0) & (xi <= wf - 1.0)
        vy = (yi >= 0.0) & (yi <= wf - 1.0)
        xc = jnp.clip(xi, 0.0, wf - 1.0).astype(jnp.int32)
        yc = jnp.clip(yi, 0.0, wf - 1.0).astype(jnp.int32)
        s_flat = basev + yc * wi + xc
        row = (n * _S + s_flat) * _M + m_idx
        wcomb = wgt * attn * vx.astype(jnp.float32) * vy.astype(jnp.float32)
        return row, wcomb

    r00, w00 = corner(x0, y0, (1.0 - fx) * (1.0 - fy))
    r10, w10 = corner(x0 + 1.0, y0, fx * (1.0 - fy))
    r01, w01 = corner(x0, y0 + 1.0, (1.0 - fx) * fy)
    r11, w11 = corner(x0 + 1.0, y0 + 1.0, fx * fy)
    idx_ref[0, 0] = jnp.concatenate([r00, r10, r01, r11], axis=-1)
    w_ref[0, 0] = jnp.concatenate([w00, w10, w01, w11], axis=-1)


def _sampling_params(queries, query_geometries, w_attn, b_attn, w_off, b_off):
    mlk = _M * _LK
    gmat = (lax.broadcasted_iota(jnp.int32, (mlk, mlk), 0) // _LK
            == lax.broadcasted_iota(jnp.int32, (mlk, mlk), 1) // _LK
            ).astype(jnp.float32)
    idx, wgt = pl.pallas_call(
        _params_body,
        grid=(_N, _NQB),
        in_specs=[
            pl.BlockSpec((1, _QB, _D), lambda n, qb: (n, qb, 0)),
            pl.BlockSpec((1, _QB, 4), lambda n, qb: (n, qb, 0)),
            pl.BlockSpec((_D, mlk), lambda n, qb: (0, 0)),
            pl.BlockSpec((1, mlk), lambda n, qb: (0, 0)),
            pl.BlockSpec((_D, mlk), lambda n, qb: (0, 0)),
            pl.BlockSpec((1, mlk), lambda n, qb: (0, 0)),
            pl.BlockSpec((_D, mlk), lambda n, qb: (0, 0)),
            pl.BlockSpec((1, mlk), lambda n, qb: (0, 0)),
            pl.BlockSpec((mlk, mlk), lambda n, qb: (0, 0)),
        ],
        out_specs=[
            pl.BlockSpec((1, 1, _QB, 4 * mlk), lambda n, qb: (n, qb, 0, 0)),
            pl.BlockSpec((1, 1, _QB, 4 * mlk), lambda n, qb: (n, qb, 0, 0)),
        ],
        out_shape=[
            jax.ShapeDtypeStruct((_N, _NQB, _QB, 4 * mlk), jnp.int32),
            jax.ShapeDtypeStruct((_N, _NQB, _QB, 4 * mlk), jnp.float32),
        ],
    )(queries, query_geometries, w_attn, b_attn.reshape(1, -1),
      w_off[:, 0::2], b_off[0::2].reshape(1, -1),
      w_off[:, 1::2], b_off[1::2].reshape(1, -1), gmat)

    return (idx.reshape(_N, _Q * 4, _M * _LK),
            wgt.reshape(_N, _Q * 4, _M * _LK))


def _sc_gather(table, cidx, cw):
    mesh = plsc.VectorSubcoreMesh(core_axis_name="c", subcore_axis_name="s")
    rpc = _QC * 4
    qpw = _Q // 8

    @functools.partial(
        pl.kernel, mesh=mesh,
        compiler_params=pltpu.CompilerParams(use_tc_tiling_on_sc=False),
        out_type=jax.ShapeDtypeStruct((_N, _Q, _M, _DV), jnp.float32),
        scratch_types=[
            pltpu.VMEM((rpc, 128), jnp.int32),
            pltpu.VMEM((rpc, 128), jnp.int32),
            pltpu.VMEM((rpc, 128), jnp.float32),
            pltpu.VMEM((rpc, 128), jnp.float32),
            pltpu.VMEM((rpc, 128, _DV), jnp.float32),
            pltpu.VMEM((rpc, 128, _DV), jnp.float32),
            pltpu.VMEM((_QC, _M, _DV), jnp.float32),
            pltpu.SemaphoreType.DMA,
            pltpu.SemaphoreType.DMA,
        ])
    def body(table_h, cidx_h, cw_h, agg_h, idx_a, idx_b, w_a, w_b, rows_a,
             rows_b, out_v, sem_a, sem_b):
        wid = lax.axis_index("s") * 2 + lax.axis_index("c")
        n = wid // 8
        q0 = lax.rem(wid, 8) * qpw

        def fire(jb, idx_v, w_v, rows_v, sem):
            r0 = (q0 + jb * _QC) * 4
            pltpu.sync_copy(cidx_h.at[n, pl.ds(r0, rpc)], idx_v)
            pltpu.sync_copy(cw_h.at[n, pl.ds(r0, rpc)], w_v)
            for j in range(rpc):
                pltpu.async_copy(table_h.at[idx_v.at[j]], rows_v.at[j], sem)

        def drain(rows_v, sem):
            for j in range(rpc):
                pltpu.make_async_copy(
                    table_h.at[pl.ds(0, 128)], rows_v.at[j], sem).wait()

        def compute(jb, w_v, rows_v):
            for q in range(_QC):
                def mloop(mi, c2, q=q):
                    cb = mi * 16
                    acc0 = jnp.zeros((16,), jnp.float32)
                    acc1 = jnp.zeros((16,), jnp.float32)
                    for c4 in range(4):
                        r = q * 4 + c4
                        wvec = w_v[r, pl.ds(cb, 16)]
                        for c16 in range(16):
                            wsc = wvec[c16]
                            acc0 = acc0 + wsc * rows_v[r, cb + c16, pl.ds(0, 16)]
                            acc1 = acc1 + wsc * rows_v[r, cb + c16, pl.ds(16, 16)]
                    out_v[q, mi, pl.ds(0, 16)] = acc0
                    out_v[q, mi, pl.ds(16, 16)] = acc1
                    return c2

                lax.fori_loop(0, _M, mloop, 0, unroll=2)
            pltpu.sync_copy(out_v, agg_h.at[n, pl.ds(q0 + jb * _QC, _QC)])

        fire(0, idx_a, w_a, rows_a, sem_a)

        def loop2(i, carry):
            jb0 = i * 2
            jb1 = jb0 + 1
            fire(jb1, idx_b, w_b, rows_b, sem_b)
            drain(rows_a, sem_a)
            compute(jb0, w_a, rows_a)

            @pl.when(jb1 + 1 < _NCH)
            def _():
                fire(jb1 + 1, idx_a, w_a, rows_a, sem_a)

            drain(rows_b, sem_b)
            compute(jb1, w_b, rows_b)
            return carry

        lax.fori_loop(0, _NCH // 2, loop2, 0)

    return body(table, cidx, cw)


def kernel(queries, query_geometries, value_inputs, value_pyramid_hw_sizes,
           W_off, b_off, W_attn, b_attn, W_val, b_val, W_out, b_out):
    del value_pyramid_hw_sizes
    proj = _matmul_bias(value_inputs.reshape(_N * _S, _D), W_val, b_val)
    table = proj.reshape(_N * _S * _M, _DV)
    cidx, cw = _sampling_params(queries, query_geometries, W_attn, b_attn,
                                W_off, b_off)
    agg = _sc_gather(table, cidx, cw)
    out = _matmul_bias(agg.reshape(_N * _Q, _M * _DV), W_out, b_out)
    return out.reshape(_N, _Q, _D)
